# baseline (speedup 1.0000x reference)
import functools

import jax
import jax.numpy as jnp
from jax import lax
from jax.experimental import pallas as pl
from jax.experimental.pallas import tpu as pltpu
from jax.experimental.pallas import tpu_sc as plsc

_CHUNK = 128


def _sc_gather(n_rows, emb_dim, dtype, n_workers, chunks_per_w):
    mesh = plsc.VectorSubcoreMesh(core_axis_name="c", subcore_axis_name="s")
    nc = mesh.num_cores

    K = 4
    G = chunks_per_w // K
    grp = K * _CHUNK

    @functools.partial(
        pl.kernel,
        out_type=jax.ShapeDtypeStruct((n_rows, 128), dtype),
        mesh=mesh,
        compiler_params=pltpu.CompilerParams(use_tc_tiling_on_sc=False),
        scratch_types=[
            pltpu.VMEM((chunks_per_w, _CHUNK), jnp.int32),
            pltpu.VMEM((2, grp, emb_dim), dtype),
            pltpu.SemaphoreType.DMA,
            pltpu.SemaphoreType.DMA,
            pltpu.SemaphoreType.DMA,
            pltpu.SemaphoreType.DMA,
        ],
    )
    def gather_k(table_hbm, idx_hbm, out_hbm, idx_v, rows_v,
                 gsem0, gsem1, ssem0, ssem1):
        wid = lax.axis_index("s") * nc + lax.axis_index("c")
        row_base = wid * chunks_per_w * _CHUNK
        gsems = (gsem0, gsem1)
        ssems = (ssem0, ssem1)
        pltpu.sync_copy(idx_hbm.at[pl.ds(wid * chunks_per_w, chunks_per_w)],
                        idx_v)

        def fire(g, b):
            for j in range(K):
                pltpu.async_copy(
                    table_hbm.at[idx_v.at[g * K + j]],
                    rows_v.at[b, pl.ds(j * _CHUNK, _CHUNK)],
                    gsems[b])

        def out_slice(g, col):
            return out_hbm.at[pl.ds(row_base + g * grp, grp),
                              pl.ds(col, emb_dim)]

        def drain(g, b):
            pltpu.make_async_copy(out_slice(g, 0), rows_v.at[b],
                                  gsems[b]).wait()

        def write(g, b):
            pltpu.async_copy(rows_v.at[b], out_slice(g, 0), ssems[b])
            pltpu.async_copy(rows_v.at[b], out_slice(g, emb_dim), ssems[b])

        def wait_write(g, b):
            pltpu.make_async_copy(rows_v.at[b], out_slice(g, 0),
                                  ssems[b]).wait()
            pltpu.make_async_copy(rows_v.at[b], out_slice(g, emb_dim),
                                  ssems[b]).wait()

        fire(0, 0)

        def body(jj, _):
            for u in range(2):
                g = jj + u
                b = u
                drain(g, b)

                @pl.when(g + 1 < G)
                def _():
                    @pl.when(g >= 1)
                    def _():
                        wait_write(g - 1, 1 - b)

                    fire(g + 1, 1 - b)

                write(g, b)
            return 0

        lax.fori_loop(0, G // 2, lambda jj, c: body(jj * 2, c), 0)
        wait_write(G - 2, 0)
        wait_write(G - 1, 1)

    return gather_k


def _tc_rnn(T, B, E, H, C):
    def body(hin_ref, x_ref, wih_ref, whh_ref, bih_ref, bhh_ref, h_ref):
        t = pl.program_id(0)

        @pl.when(t == 0)
        def _():
            h_ref[...] = hin_ref[...]

        x = x_ref[...].astype(jnp.bfloat16)
        h = h_ref[...].astype(jnp.bfloat16)
        h_ref[...] = jnp.tanh(
            jnp.dot(x, wih_ref[...], preferred_element_type=jnp.float32)
            + jnp.dot(h, whh_ref[...], preferred_element_type=jnp.float32)
            + bih_ref[...] + bhh_ref[...])

    scan = pl.pallas_call(
        body,
        grid=(T,),
        in_specs=[
            pl.BlockSpec((B, H), lambda t: (0, 0)),
            pl.BlockSpec((B, 128), lambda t: (t, 0)),
            pl.BlockSpec((128, H), lambda t: (0, 0)),
            pl.BlockSpec((H, H), lambda t: (0, 0)),
            pl.BlockSpec((1, H), lambda t: (0, 0)),
            pl.BlockSpec((1, H), lambda t: (0, 0)),
        ],
        out_specs=pl.BlockSpec((B, H), lambda t: (0, 0)),
        out_shape=jax.ShapeDtypeStruct((B, H), jnp.float32),
    )

    def head_body(h_ref, wlin_ref, blin_ref, out_ref):
        logits = jnp.dot(h_ref[...], wlin_ref[...],
                         preferred_element_type=jnp.float32) + blin_ref[...]
        m = jnp.max(logits, axis=1, keepdims=True)
        e = jnp.exp(logits - m)
        out_ref[...] = e / jnp.sum(e, axis=1, keepdims=True)

    head = pl.pallas_call(
        head_body,
        out_shape=jax.ShapeDtypeStruct((B, C), jnp.float32),
    )
    return scan, head


def kernel(text, emb, W_ih, W_hh, b_ih, b_hh, W_lin, b_lin):
    B, T = text.shape
    V, E = emb.shape
    H = W_hh.shape[0]
    C = W_lin.shape[0]

    n_workers = 32
    if T == 200:
        sizes = (8, 24, 56, 56, 56)
    else:
        sizes = (T,)
    idx_flat = text.T.reshape(-1).astype(jnp.int32)

    wih = jnp.concatenate(
        [W_ih.T, jnp.zeros((128 - E, H), W_ih.dtype)]).astype(jnp.bfloat16)
    whh = W_hh.T.astype(jnp.bfloat16)
    bih = b_ih.reshape(1, H)
    bhh = b_hh.reshape(1, H)

    h = jnp.zeros((B, H), jnp.float32)
    t0 = 0
    scans = {}
    gathers = {}
    for Tp in sizes:
        np_rows = Tp * B
        chunks_per_w = np_rows // (n_workers * _CHUNK)
        idx_p = idx_flat[t0 * B:(t0 + Tp) * B].reshape(-1, _CHUNK)
        if Tp not in gathers:
            gathers[Tp] = _sc_gather(np_rows, E, emb.dtype, n_workers,
                                     chunks_per_w)
            scans[Tp] = _tc_rnn(Tp, B, E, H, C)[0]
        x_p = gathers[Tp](emb, idx_p)
        h = scans[Tp](h, x_p, wih, whh, bih, bhh)
        t0 += Tp
    head = _tc_rnn(T, B, E, H, C)[1]
    return head(h, W_lin.T, b_lin.reshape(1, C))


# speedup vs baseline: 1.1292x; 1.1292x over previous
import functools

import jax
import jax.numpy as jnp
from jax import lax
from jax.experimental import pallas as pl
from jax.experimental.pallas import tpu as pltpu
from jax.experimental.pallas import tpu_sc as plsc

_CHUNK = 128


def _sc_gather(n_rows, emb_dim, dtype, n_workers, chunks_per_w):
    mesh = plsc.VectorSubcoreMesh(core_axis_name="c", subcore_axis_name="s")
    nc = mesh.num_cores

    K = 4
    G = chunks_per_w // K
    grp = K * _CHUNK

    @functools.partial(
        pl.kernel,
        out_type=jax.ShapeDtypeStruct((n_rows, 128), dtype),
        mesh=mesh,
        compiler_params=pltpu.CompilerParams(use_tc_tiling_on_sc=False),
        scratch_types=[
            pltpu.VMEM((chunks_per_w, _CHUNK), jnp.int32),
            pltpu.VMEM((2, grp, emb_dim), dtype),
            pltpu.SemaphoreType.DMA,
            pltpu.SemaphoreType.DMA,
            pltpu.SemaphoreType.DMA,
            pltpu.SemaphoreType.DMA,
        ],
    )
    def gather_k(table_hbm, idx_hbm, out_hbm, idx_v, rows_v,
                 gsem0, gsem1, ssem0, ssem1):
        wid = lax.axis_index("s") * nc + lax.axis_index("c")
        row_base = wid * chunks_per_w * _CHUNK
        gsems = (gsem0, gsem1)
        ssems = (ssem0, ssem1)
        pltpu.sync_copy(idx_hbm.at[pl.ds(wid * chunks_per_w, chunks_per_w)],
                        idx_v)

        def fire(g, b):
            for j in range(K):
                pltpu.async_copy(
                    table_hbm.at[idx_v.at[g * K + j]],
                    rows_v.at[b, pl.ds(j * _CHUNK, _CHUNK)],
                    gsems[b])

        def out_slice(g, col):
            return out_hbm.at[pl.ds(row_base + g * grp, grp),
                              pl.ds(col, emb_dim)]

        def drain(g, b):
            pltpu.make_async_copy(out_slice(g, 0), rows_v.at[b],
                                  gsems[b]).wait()

        def write(g, b):
            pltpu.async_copy(rows_v.at[b], out_slice(g, 0), ssems[b])

        def wait_write(g, b):
            pltpu.make_async_copy(rows_v.at[b], out_slice(g, 0),
                                  ssems[b]).wait()

        fire(0, 0)

        def body(jj, _):
            for u in range(2):
                g = jj + u
                b = u
                drain(g, b)

                @pl.when(g + 1 < G)
                def _():
                    @pl.when(g >= 1)
                    def _():
                        wait_write(g - 1, 1 - b)

                    fire(g + 1, 1 - b)

                write(g, b)
            return 0

        lax.fori_loop(0, G // 2, lambda jj, c: body(jj * 2, c), 0)
        wait_write(G - 2, 0)
        wait_write(G - 1, 1)

    return gather_k


def _tc_rnn(T, B, E, H, C):
    def body(hin_ref, x_ref, wih_ref, whh_ref, bih_ref, bhh_ref, h_ref):
        t = pl.program_id(0)

        @pl.when(t == 0)
        def _():
            h_ref[...] = hin_ref[...]

        lane = lax.broadcasted_iota(jnp.int32, (B, 128), 1)
        x = jnp.where(lane < E, x_ref[...], 0.0).astype(jnp.bfloat16)
        h = h_ref[...].astype(jnp.bfloat16)
        h_ref[...] = jnp.tanh(
            jnp.dot(x, wih_ref[...], preferred_element_type=jnp.float32)
            + jnp.dot(h, whh_ref[...], preferred_element_type=jnp.float32)
            + bih_ref[...] + bhh_ref[...])

    scan = pl.pallas_call(
        body,
        grid=(T,),
        in_specs=[
            pl.BlockSpec((B, H), lambda t: (0, 0)),
            pl.BlockSpec((B, 128), lambda t: (t, 0)),
            pl.BlockSpec((128, H), lambda t: (0, 0)),
            pl.BlockSpec((H, H), lambda t: (0, 0)),
            pl.BlockSpec((1, H), lambda t: (0, 0)),
            pl.BlockSpec((1, H), lambda t: (0, 0)),
        ],
        out_specs=pl.BlockSpec((B, H), lambda t: (0, 0)),
        out_shape=jax.ShapeDtypeStruct((B, H), jnp.float32),
    )

    def head_body(h_ref, wlin_ref, blin_ref, out_ref):
        logits = jnp.dot(h_ref[...], wlin_ref[...],
                         preferred_element_type=jnp.float32) + blin_ref[...]
        m = jnp.max(logits, axis=1, keepdims=True)
        e = jnp.exp(logits - m)
        out_ref[...] = e / jnp.sum(e, axis=1, keepdims=True)

    head = pl.pallas_call(
        head_body,
        out_shape=jax.ShapeDtypeStruct((B, C), jnp.float32),
    )
    return scan, head


def kernel(text, emb, W_ih, W_hh, b_ih, b_hh, W_lin, b_lin):
    B, T = text.shape
    V, E = emb.shape
    H = W_hh.shape[0]
    C = W_lin.shape[0]

    n_workers = 32
    if T == 200:
        sizes = (8, 24, 56, 56, 56)
    else:
        sizes = (T,)
    wih = jnp.concatenate(
        [W_ih.T, jnp.zeros((128 - E, H), W_ih.dtype)]).astype(jnp.bfloat16)
    whh = W_hh.T.astype(jnp.bfloat16)
    bih = b_ih.reshape(1, H)
    bhh = b_hh.reshape(1, H)

    h = jnp.zeros((B, H), jnp.float32)
    t0 = 0
    scans = {}
    gathers = {}
    for Tp in sizes:
        np_rows = Tp * B
        chunks_per_w = np_rows // (n_workers * _CHUNK)
        idx_p = text[:, t0:t0 + Tp].T.reshape(-1, _CHUNK).astype(jnp.int32)
        if Tp not in gathers:
            gathers[Tp] = _sc_gather(np_rows, E, emb.dtype, n_workers,
                                     chunks_per_w)
            scans[Tp] = _tc_rnn(Tp, B, E, H, C)[0]
        x_p = gathers[Tp](emb, idx_p)
        h = scans[Tp](h, x_p, wih, whh, bih, bhh)
        t0 += Tp
    head = _tc_rnn(T, B, E, H, C)[1]
    return head(h, W_lin.T, b_lin.reshape(1, C))
